# baseline (speedup 1.0000x reference)
import functools
import math

import jax
import jax.numpy as jnp
from jax import lax
from jax.experimental import pallas as pl
from jax.experimental.pallas import tpu as pltpu
from jax.experimental.pallas import tpu_sc as plsc

_LEN_MAX = 4096


def _pe_table(S, D, dtype):
    t = (jnp.arange(S + 1, dtype=dtype) - 1)[:, None]
    j = jnp.arange(D // 2, dtype=dtype)[None, :]
    k = j % 2
    return jnp.sin(t / (_LEN_MAX ** ((j - k) / D)) + (math.pi / 2) * k)


@functools.lru_cache(maxsize=None)
def _make_sc_kernel(N, D, V):
    H = D // 2
    NW = 32
    rows_per_w = N // NW
    R = 8
    NSLOT = 8
    NPE = 8
    SH = (H // R).bit_length() - 1
    n_chunks = rows_per_w // R
    n_iter = n_chunks // NSLOT
    nv32 = H // 32

    mesh = plsc.VectorSubcoreMesh(core_axis_name="c", subcore_axis_name="s")

    scratch = (
        [pltpu.VMEM((R, D), jnp.float32) for _ in range(NSLOT)]
        + [pltpu.VMEM((2 * R, H // 2), jnp.int32) for _ in range(NPE)]
        + [pltpu.VMEM((2 * (N // NW),), jnp.int32)]
        + [pltpu.SemaphoreType.DMA for _ in range(2 * NSLOT + NPE)]
    )

    @functools.partial(
        pl.kernel,
        out_type=jax.ShapeDtypeStruct((N, D), jnp.float32),
        mesh=mesh,
        scratch_types=scratch,
    )
    def k(x_hbm, idx_hbm, pe_hbm, out_hbm, *refs):
        xslots = refs[0:NSLOT]
        pes = refs[NSLOT:NSLOT + NPE]
        idxall = refs[NSLOT + NPE]
        sems = refs[NSLOT + NPE + 1:]
        semins = sems[0:NSLOT]
        semouts = sems[NSLOT:2 * NSLOT]
        sempes = sems[2 * NSLOT:]

        wid = lax.axis_index("s") * 2 + lax.axis_index("c")
        base0 = wid * rows_per_w

        def issue_in(c, xs, ps):
            base = base0 + c * R
            pltpu.async_copy(x_hbm.at[pl.ds(base, R)], xslots[xs], semins[xs])
            pltpu.async_copy(pe_hbm.at[idxall.at[pl.ds(2 * c * R, 2 * R)]],
                             pes[ps], sempes[ps])

        def wait_in(xs, ps):
            pltpu.make_async_copy(
                x_hbm.at[pl.ds(0, R)], xslots[xs], semins[xs]).wait()
            pltpu.make_async_copy(
                pe_hbm.at[pl.ds(0, 2 * R)], pes[ps], sempes[ps]).wait()

        def issue_out(c, xs):
            base = base0 + c * R
            pltpu.async_copy(xslots[xs], out_hbm.at[pl.ds(base, R)],
                             semouts[xs])

        def wait_out(xs):
            pltpu.make_async_copy(
                xslots[xs], out_hbm.at[pl.ds(0, R)], semouts[xs]).wait()

        def compute(xs, ps):
            xb, pe_ = xslots[xs], pes[ps]

            @plsc.parallel_loop(0, 2 * R * nv32, unroll=8)
            def _vec(i):
                rr = lax.shift_right_logical(i, 4)
                r = lax.bitwise_and(rr, R - 1)
                half = lax.shift_left(lax.bitwise_and(rr, R), SH)
                v2 = lax.shift_left(lax.bitwise_and(i, nv32 - 1), 4)
                w = pe_[rr, pl.ds(pl.multiple_of(v2, 16), 16)]
                a = lax.bitcast_convert_type(
                    lax.shift_left(w, 16), jnp.float32)
                b = lax.bitcast_convert_type(
                    lax.bitwise_and(w, jnp.int32(-65536)), jnp.float32)
                o = pl.multiple_of(half + 2 * v2, 16)
                plsc.addupdate(xb.at[r, pl.ds(o, 16)], a)
                plsc.addupdate(xb.at[r, pl.ds(o + 16, 16)], b)

        pltpu.sync_copy(idx_hbm.at[pl.ds(2 * base0, 2 * rows_per_w)], idxall)
        for p in range(NSLOT - 2):
            issue_in(jnp.int32(p), p, p)

        def body(q, carry):
            c0 = q * NSLOT
            for j in range(NSLOT):
                c = c0 + j
                wait_in(j, j)
                t = (j + NSLOT - 2) % NSLOT
                if j < 2:
                    @pl.when(q > 0)
                    def _():
                        wait_out(t)

                    issue_in(c + NSLOT - 2, t, t)
                else:
                    @pl.when(q < n_iter - 1)
                    def _():
                        wait_out(t)
                        issue_in(c + NSLOT - 2, t, t)
                compute(j, j)
                issue_out(c, j)
            return carry

        lax.fori_loop(0, n_iter, body, 0)
        for s in range(NSLOT):
            wait_out(s)

    return k


def kernel(x, order):
    B, S, D = x.shape
    N, H, R = B * S, D // 2, 8
    pe = _pe_table(S, D, x.dtype)
    pe16 = (pe.astype(jnp.bfloat16)
            .reshape(S + 1, H // 32, 2, 16)
            .transpose(0, 1, 3, 2)
            .reshape(S + 1, H // 2, 2))
    pe16 = lax.bitcast_convert_type(pe16, jnp.int32)
    idx_out = (order + 1).astype(jnp.int32)
    idx_in = jnp.pad(idx_out, ((0, 0), (1, 0)))[:, :S]
    idxcat = jnp.concatenate(
        [idx_in.reshape(N // R, R), idx_out.reshape(N // R, R)], axis=1
    ).reshape(-1)
    sc = _make_sc_kernel(N, D, S + 1)
    out = sc(x.reshape(N, D), idxcat, pe16)
    return out.reshape(B, S, D)


# speedup vs baseline: 1.0140x; 1.0140x over previous
import functools
import math

import jax
import jax.numpy as jnp
from jax import lax
from jax.experimental import pallas as pl
from jax.experimental.pallas import tpu as pltpu
from jax.experimental.pallas import tpu_sc as plsc

_LEN_MAX = 4096


def _pe_table(S, D, dtype):
    t = (jnp.arange(S + 1, dtype=dtype) - 1)[:, None]
    j = jnp.arange(D // 2, dtype=dtype)[None, :]
    k = j % 2
    return jnp.sin(t / (_LEN_MAX ** ((j - k) / D)) + (math.pi / 2) * k)


@functools.lru_cache(maxsize=None)
def _make_sc_kernel(N, D, V):
    H = D // 2
    NW = 32
    rows_per_w = N // NW
    R = 16
    NSLOT = 4
    NPE = 4
    n_chunks = rows_per_w // R
    n_iter = n_chunks // NSLOT
    nv32 = H // 32

    mesh = plsc.VectorSubcoreMesh(core_axis_name="c", subcore_axis_name="s")

    scratch = (
        [pltpu.VMEM((R, D), jnp.float32) for _ in range(NSLOT)]
        + [pltpu.VMEM((2 * R, H // 2), jnp.int32) for _ in range(NPE)]
        + [pltpu.VMEM((2 * (N // NW),), jnp.int32)]
        + [pltpu.SemaphoreType.DMA for _ in range(2 * NSLOT + NPE)]
    )

    @functools.partial(
        pl.kernel,
        out_type=jax.ShapeDtypeStruct((N, D), jnp.float32),
        mesh=mesh,
        scratch_types=scratch,
    )
    def k(x_hbm, idx_hbm, pe_hbm, out_hbm, *refs):
        xslots = refs[0:NSLOT]
        pes = refs[NSLOT:NSLOT + NPE]
        idxall = refs[NSLOT + NPE]
        sems = refs[NSLOT + NPE + 1:]
        semins = sems[0:NSLOT]
        semouts = sems[NSLOT:2 * NSLOT]
        sempes = sems[2 * NSLOT:]

        wid = lax.axis_index("s") * 2 + lax.axis_index("c")
        base0 = wid * rows_per_w

        def issue_in(c, xs, ps):
            base = base0 + c * R
            pltpu.async_copy(x_hbm.at[pl.ds(base, R)], xslots[xs], semins[xs])
            pltpu.async_copy(pe_hbm.at[idxall.at[pl.ds(2 * c * R, 2 * R)]],
                             pes[ps], sempes[ps])

        def wait_in(xs, ps):
            pltpu.make_async_copy(
                x_hbm.at[pl.ds(0, R)], xslots[xs], semins[xs]).wait()
            pltpu.make_async_copy(
                pe_hbm.at[pl.ds(0, 2 * R)], pes[ps], sempes[ps]).wait()

        def issue_out(c, xs):
            base = base0 + c * R
            pltpu.async_copy(xslots[xs], out_hbm.at[pl.ds(base, R)],
                             semouts[xs])

        def wait_out(xs):
            pltpu.make_async_copy(
                xslots[xs], out_hbm.at[pl.ds(0, R)], semouts[xs]).wait()

        def compute(xs, ps):
            xb, pe_ = xslots[xs], pes[ps]

            @plsc.parallel_loop(0, 2 * R * nv32, unroll=8)
            def _vec(i):
                rr = lax.shift_right_logical(i, 4)
                r = lax.bitwise_and(rr, R - 1)
                half = lax.shift_left(lax.bitwise_and(rr, R), 5)
                v2 = lax.shift_left(lax.bitwise_and(i, nv32 - 1), 4)
                w = pe_[rr, pl.ds(pl.multiple_of(v2, 16), 16)]
                a = lax.bitcast_convert_type(
                    lax.shift_left(w, 16), jnp.float32)
                b = lax.bitcast_convert_type(
                    lax.bitwise_and(w, jnp.int32(-65536)), jnp.float32)
                o = pl.multiple_of(half + 2 * v2, 16)
                plsc.addupdate(xb.at[r, pl.ds(o, 16)], a)
                plsc.addupdate(xb.at[r, pl.ds(o + 16, 16)], b)

        pltpu.sync_copy(idx_hbm.at[pl.ds(2 * base0, 2 * rows_per_w)], idxall)
        issue_in(jnp.int32(0), 0, 0)
        issue_in(jnp.int32(1), 1, 1)

        def body(q, carry):
            c0 = q * NSLOT
            for j in range(NSLOT):
                c = c0 + j
                wait_in(j, j)
                t = (j + 2) % NSLOT
                if j < 2:
                    @pl.when(q > 0)
                    def _():
                        wait_out(t)

                    issue_in(c + 2, t, t)
                else:
                    @pl.when(q < n_iter - 1)
                    def _():
                        wait_out(t)
                        issue_in(c + 2, t, t)
                compute(j, j)
                issue_out(c, j)
            return carry

        lax.fori_loop(0, n_iter, body, 0)
        for s in range(NSLOT):
            wait_out(s)

    return k


def kernel(x, order):
    B, S, D = x.shape
    N, H, R = B * S, D // 2, 16
    pe = _pe_table(S, D, x.dtype)
    pe16 = (pe.astype(jnp.bfloat16)
            .reshape(S + 1, H // 32, 2, 16)
            .transpose(0, 1, 3, 2)
            .reshape(S + 1, H // 2, 2))
    pe16 = lax.bitcast_convert_type(pe16, jnp.int32)
    idx_out = (order + 1).astype(jnp.int32)
    idx_in = jnp.pad(idx_out, ((0, 0), (1, 0)))[:, :S]
    idxcat = jnp.concatenate(
        [idx_in.reshape(N // R, R), idx_out.reshape(N // R, R)], axis=1
    ).reshape(-1)
    sc = _make_sc_kernel(N, D, S + 1)
    out = sc(x.reshape(N, D), idxcat, pe16)
    return out.reshape(B, S, D)
